# Initial kernel scaffold; baseline (speedup 1.0000x reference)
import jax
import jax.numpy as jnp
from jax.experimental import pallas as pl


def kernel(section_length, item_size, cumsum_tril_value_indices, cumsum_tril_indices, Xs_clicked, disp_2d_split_sec_ind, disp_current_feature, W1, b1, W2, b2, W_out, b_out, position_weight):
    raise NotImplementedError("write your pallas kernel here")


# speedup vs baseline: 1.0895x; 1.0895x over previous
import functools

import jax
import jax.numpy as jnp
from jax import lax
from jax.experimental import pallas as pl
from jax.experimental.pallas import tpu as pltpu

F = 128
R = 512


def _mlp_body(chg_ref, disp_ref, w1_ref, b1_ref, w2_ref, b2_ref,
              wout_ref, bout_ref, out_ref):
    x = jnp.concatenate([chg_ref[...], disp_ref[...]], axis=1)
    h1 = jnp.dot(x, w1_ref[...], preferred_element_type=jnp.float32)
    h1 = h1 + b1_ref[...]
    h1 = jnp.where(h1 > 0, h1, jnp.exp(h1) - 1.0)
    h1 = h1.astype(jnp.bfloat16)
    h2 = jnp.dot(h1, w2_ref[...], preferred_element_type=jnp.float32)
    h2 = h2 + b2_ref[...]
    h2 = jnp.where(h2 > 0, h2, jnp.exp(h2) - 1.0)
    h2 = h2.astype(jnp.bfloat16)
    u = jnp.dot(h2, wout_ref[...], preferred_element_type=jnp.float32)
    u = u + bout_ref[...]
    out_ref[...] = jnp.exp(u)


def _mlp_exp(chg, disp, w1t, b1, w2t, b2, woutt, bout):
    n = chg.shape[0]
    grid = n // R
    return pl.pallas_call(
        _mlp_body,
        grid=(grid,),
        in_specs=[
            pl.BlockSpec((R, F), lambda i: (i, 0)),
            pl.BlockSpec((R, F), lambda i: (i, 0)),
            pl.BlockSpec((2 * F, 1024), lambda i: (0, 0)),
            pl.BlockSpec((1, 1024), lambda i: (0, 0)),
            pl.BlockSpec((1024, 1024), lambda i: (0, 0)),
            pl.BlockSpec((1, 1024), lambda i: (0, 0)),
            pl.BlockSpec((1024, 128), lambda i: (0, 0)),
            pl.BlockSpec((1, 128), lambda i: (0, 0)),
        ],
        out_specs=pl.BlockSpec((R, 128), lambda i: (i, 0)),
        out_shape=jax.ShapeDtypeStruct((n, 128), jnp.float32),
    )(chg, disp, w1t, b1, w2t, b2, woutt, bout)


def kernel(section_length, item_size, cumsum_tril_value_indices,
           cumsum_tril_indices, Xs_clicked, disp_2d_split_sec_ind,
           disp_current_feature, W1, b1, W2, b2, W_out, b_out,
           position_weight):
    sec_len = Xs_clicked.shape[0]
    n_disp = disp_2d_split_sec_ind.shape[0]

    rows = cumsum_tril_indices[:, 0]
    cols = cumsum_tril_indices[:, 1]
    vals = jnp.take(position_weight, cumsum_tril_value_indices, axis=0)
    ch = jnp.zeros((sec_len, F), jnp.float32).at[rows].add(
        vals[:, None] * jnp.take(Xs_clicked, cols, axis=0))
    chg = jnp.take(ch, disp_2d_split_sec_ind, axis=0)

    w1_hist = (W1[:, 0:F] + W1[:, F:2 * F] + W1[:, 2 * F:3 * F]
               + W1[:, 3 * F:4 * F])
    w1t = jnp.concatenate([w1_hist, W1[:, 4 * F:5 * F]], axis=1).T
    w1t = w1t.astype(jnp.bfloat16)
    w2t = W2.T.astype(jnp.bfloat16)
    woutt = jnp.zeros((1024, 128), jnp.bfloat16).at[:, 0].set(
        W_out[0, :].astype(jnp.bfloat16))
    boutv = jnp.zeros((1, 128), jnp.float32).at[0, 0].set(b_out[0])

    exp_u = _mlp_exp(chg.astype(jnp.bfloat16),
                     disp_current_feature.astype(jnp.bfloat16),
                     w1t, b1.reshape(1, -1), w2t, b2.reshape(1, -1),
                     woutt, boutv)
    exp_u = exp_u[:, 0:1]

    out = jax.ops.segment_sum(exp_u, disp_2d_split_sec_ind,
                              num_segments=sec_len)
    return out


# speedup vs baseline: 2.6703x; 2.4509x over previous
import functools

import jax
import jax.numpy as jnp
from jax import lax
from jax.experimental import pallas as pl
from jax.experimental.pallas import tpu as pltpu
from jax.experimental.pallas import tpu_sc as plsc

F = 128
FH = 64
R = 512
NNZ = 262144
SEC = 16384
NDISP = 65536
NS = 16


def _spmm_body(rows_h, cols_h, vidx_h, xsl_h, xsr_h, disp_h, pwx_h,
               chgl_h, chgr_h, chfl_h, chfr_h,
               rowb, colb, vib, rbuf, vbufx, idxd, acc, sem):
    cid = lax.axis_index("c")
    sid = lax.axis_index("s")

    def zrow(r, c):
        for q in range(FH // 16):
            rbuf[r, pl.ds(q * 16, 16)] = jnp.zeros((16,), jnp.float32)
        return c
    lax.fori_loop(0, 128, zrow, 0)
    for p in range(8):
        pltpu.sync_copy(rbuf, acc.at[pl.ds(sid * 1024 + p * 128, 128)])
    plsc.subcore_barrier()

    def process(xs_h, chg_h, chf_h):
        def mc_body(mc, c):
            base = sid * 128 + mc * 8
            pltpu.sync_copy(rows_h.at[pl.ds(base, 8)], rowb)
            pltpu.sync_copy(cols_h.at[pl.ds(base, 8)], colb)
            pltpu.sync_copy(vidx_h.at[pl.ds(base, 8)], vib)

            def j_body(j, c2):
                d1 = pltpu.async_copy(xs_h.at[colb.at[j]], rbuf, sem)
                d2 = pltpu.async_copy(pwx_h.at[vib.at[j]], vbufx, sem)
                d1.wait()
                d2.wait()

                def m_body(r, c3):
                    for q in range(FH // 16):
                        rbuf[r, pl.ds(q * 16, 16)] = (
                            rbuf[r, pl.ds(q * 16, 16)]
                            * vbufx[r, pl.ds(q * 16, 16)])
                    return c3
                lax.fori_loop(0, 128, m_body, 0)
                pltpu.sync_copy(rbuf, acc.at[rowb.at[j]], add=True)
                return c2
            lax.fori_loop(0, 8, j_body, 0)
            return c
        lax.fori_loop(0, 16, mc_body, 0)
        plsc.subcore_barrier()

        pltpu.sync_copy(acc.at[pl.ds(sid * 1024, 1024)],
                        chf_h.at[pl.ds(sid * 1024, 1024)])
        plsc.subcore_barrier()

        def gc_body(k, c):
            gbase = sid * 4096 + k * 128
            pltpu.sync_copy(disp_h.at[pl.ds(gbase, 128)], idxd)
            pltpu.async_copy(chf_h.at[idxd], rbuf, sem).wait()
            pltpu.sync_copy(rbuf, chg_h.at[pl.ds(gbase, 128)])
            return c
        lax.fori_loop(0, 32, gc_body, 0)

    pl.when(cid == 0)(lambda: process(xsl_h, chgl_h, chfl_h))
    pl.when(cid == 1)(lambda: process(xsr_h, chgr_h, chfr_h))


@functools.partial(jax.jit, static_argnums=())
def _spmm_gather(rows2, cols2, vidx2, xsl, xsr, dispi, pw_pad):
    mesh = plsc.VectorSubcoreMesh(core_axis_name="c", subcore_axis_name="s")
    f = pl.kernel(
        _spmm_body,
        out_type=[jax.ShapeDtypeStruct((NDISP, FH), jnp.float32),
                  jax.ShapeDtypeStruct((NDISP, FH), jnp.float32),
                  jax.ShapeDtypeStruct((SEC, FH), jnp.float32),
                  jax.ShapeDtypeStruct((SEC, FH), jnp.float32)],
        mesh=mesh,
        scratch_types=[
            pltpu.VMEM((8, 128), jnp.int32),
            pltpu.VMEM((8, 128), jnp.int32),
            pltpu.VMEM((8, 128), jnp.int32),
            pltpu.VMEM((128, FH), jnp.float32),
            pltpu.VMEM((128, FH), jnp.float32),
            pltpu.VMEM((128,), jnp.int32),
            pltpu.VMEM_SHARED((SEC, FH), jnp.float32),
            pltpu.SemaphoreType.DMA,
        ],
        compiler_params=pltpu.CompilerParams(needs_layout_passes=False,
                                             use_tc_tiling_on_sc=False),
    )
    chgl, chgr, _, _ = f(rows2, cols2, vidx2, xsl, xsr, dispi, pw_pad)
    return chgl, chgr


def _seg_body(exp_h, disp_h, zeros_h, out_h, ibuf, dbuf, sacc, sem):
    cid = lax.axis_index("c")
    sid = lax.axis_index("s")

    @pl.when(cid == 0)
    def _():
        pltpu.sync_copy(zeros_h.at[pl.ds(sid * 1024, 1024)],
                        sacc.at[pl.ds(sid * 1024, 1024)])
        plsc.subcore_barrier()

        def sc_body(k, c):
            gbase = sid * 4096 + k * 128
            pltpu.sync_copy(disp_h.at[pl.ds(gbase, 128)], ibuf.at[0])
            pltpu.sync_copy(exp_h.at[pl.ds(gbase, 128)], dbuf)
            pltpu.sync_copy(dbuf, sacc.at[ibuf.at[0]], add=True)
            return c
        lax.fori_loop(0, 32, sc_body, 0)
        plsc.subcore_barrier()
        pltpu.sync_copy(sacc.at[pl.ds(sid * 1024, 1024)],
                        out_h.at[pl.ds(sid * 1024, 1024)])


def _segsum(exp8, dispi, zeros8):
    mesh = plsc.VectorSubcoreMesh(core_axis_name="c", subcore_axis_name="s")
    f = pl.kernel(
        _seg_body,
        out_type=jax.ShapeDtypeStruct((SEC, 8), jnp.float32),
        mesh=mesh,
        scratch_types=[
            pltpu.VMEM((1, 128), jnp.int32),
            pltpu.VMEM((128, 8), jnp.float32),
            pltpu.VMEM_SHARED((SEC, 8), jnp.float32),
            pltpu.SemaphoreType.DMA,
        ],
        compiler_params=pltpu.CompilerParams(needs_layout_passes=False,
                                             use_tc_tiling_on_sc=False),
    )
    return f(exp8, dispi, zeros8)


def _mlp_body(chgl_ref, chgr_ref, disp_ref, w1_ref, b1_ref, w2_ref, b2_ref,
              wout_ref, bout_ref, out_ref):
    x = jnp.concatenate(
        [chgl_ref[...].astype(jnp.bfloat16),
         chgr_ref[...].astype(jnp.bfloat16),
         disp_ref[...]], axis=1)
    h1 = jnp.dot(x, w1_ref[...], preferred_element_type=jnp.float32)
    h1 = h1 + b1_ref[...]
    h1 = jnp.where(h1 > 0, h1, jnp.exp(h1) - 1.0)
    h1 = h1.astype(jnp.bfloat16)
    h2 = jnp.dot(h1, w2_ref[...], preferred_element_type=jnp.float32)
    h2 = h2 + b2_ref[...]
    h2 = jnp.where(h2 > 0, h2, jnp.exp(h2) - 1.0)
    h2 = h2.astype(jnp.bfloat16)
    u = jnp.dot(h2, wout_ref[...], preferred_element_type=jnp.float32)
    u = u + bout_ref[...]
    out_ref[...] = jnp.exp(u[:, :8])


def _mlp_exp(chgl, chgr, disp, w1t, b1, w2t, b2, woutt, bout):
    n = disp.shape[0]
    grid = n // R
    return pl.pallas_call(
        _mlp_body,
        grid=(grid,),
        in_specs=[
            pl.BlockSpec((R, FH), lambda i: (i, 0)),
            pl.BlockSpec((R, FH), lambda i: (i, 0)),
            pl.BlockSpec((R, F), lambda i: (i, 0)),
            pl.BlockSpec((2 * F, 1024), lambda i: (0, 0)),
            pl.BlockSpec((1, 1024), lambda i: (0, 0)),
            pl.BlockSpec((1024, 1024), lambda i: (0, 0)),
            pl.BlockSpec((1, 1024), lambda i: (0, 0)),
            pl.BlockSpec((1024, 128), lambda i: (0, 0)),
            pl.BlockSpec((1, 128), lambda i: (0, 0)),
        ],
        out_specs=pl.BlockSpec((R, 8), lambda i: (i, 0)),
        out_shape=jax.ShapeDtypeStruct((n, 8), jnp.float32),
    )(chgl, chgr, disp, w1t, b1, w2t, b2, woutt, bout)


def kernel(section_length, item_size, cumsum_tril_value_indices,
           cumsum_tril_indices, Xs_clicked, disp_2d_split_sec_ind,
           disp_current_feature, W1, b1, W2, b2, W_out, b_out,
           position_weight):
    rows2 = cumsum_tril_indices[:, 0].reshape(NNZ // 128, 128)
    cols2 = cumsum_tril_indices[:, 1].reshape(NNZ // 128, 128)
    vidx2 = cumsum_tril_value_indices.reshape(NNZ // 128, 128)
    xsl = Xs_clicked[:, :FH]
    xsr = Xs_clicked[:, FH:]
    pw_pad = jnp.zeros((64,), jnp.float32).at[:50].set(position_weight)
    pw_exp = jnp.broadcast_to(pw_pad[:, None], (64, FH))
    dispi = disp_2d_split_sec_ind.astype(jnp.int32)

    chgl, chgr = _spmm_gather(rows2, cols2, vidx2, xsl, xsr, dispi, pw_exp)

    w1_hist = (W1[:, 0:F] + W1[:, F:2 * F] + W1[:, 2 * F:3 * F]
               + W1[:, 3 * F:4 * F])
    w1t = jnp.concatenate([w1_hist, W1[:, 4 * F:5 * F]], axis=1).T
    w1t = w1t.astype(jnp.bfloat16)
    w2t = W2.T.astype(jnp.bfloat16)
    woutt = jnp.zeros((1024, 128), jnp.bfloat16).at[:, 0].set(
        W_out[0, :].astype(jnp.bfloat16))
    boutv = jnp.zeros((1, 128), jnp.float32).at[0, 0].set(b_out[0])

    exp8 = _mlp_exp(chgl, chgr, disp_current_feature.astype(jnp.bfloat16),
                    w1t, b1.reshape(1, -1), w2t, b2.reshape(1, -1),
                    woutt, boutv)

    zeros8 = jnp.zeros((SEC, 8), jnp.float32)
    out8 = _segsum(exp8, dispi, zeros8)
    return out8[:, 0:1]


# speedup vs baseline: 2.6955x; 1.0094x over previous
import functools

import jax
import jax.numpy as jnp
from jax import lax
from jax.experimental import pallas as pl
from jax.experimental.pallas import tpu as pltpu
from jax.experimental.pallas import tpu_sc as plsc

F = 128
FH = 64
R = 512
NNZ = 262144
SEC = 16384
NDISP = 65536
NS = 16


NBUF = 2


def _spmm_body(rows_h, cols_h, vidx_h, xsl_h, xsr_h, disp_h, pwx_h,
               chgl_h, chgr_h, chfl_h, chfr_h,
               rowb, colb, vib, dispb,
               rb0, rb1, vb0, vb1,
               acc, sem0, sem1):
    cid = lax.axis_index("c")
    sid = lax.axis_index("s")
    bufs = ((rb0, vb0, sem0), (rb1, vb1, sem1))

    def zrow(r, c):
        for q in range(FH // 16):
            rb0[r, pl.ds(q * 16, 16)] = jnp.zeros((16,), jnp.float32)
        return c
    lax.fori_loop(0, 128, zrow, 0)
    for p in range(8):
        pltpu.sync_copy(rb0, acc.at[pl.ds(sid * 1024 + p * 128, 128)])
    plsc.subcore_barrier()

    def process(xs_h, chg_h, chf_h):
        def macro(m, c0):
            base = sid * 128 + m * 32
            pltpu.sync_copy(rows_h.at[pl.ds(base, 32)], rowb)
            pltpu.sync_copy(cols_h.at[pl.ds(base, 32)], colb)
            pltpu.sync_copy(vidx_h.at[pl.ds(base, 32)], vib)

            def k_body(k, c):
                gds = []
                for b in range(NBUF):
                    rbx, vbx, semx = bufs[b]
                    ck = k * NBUF + b
                    gds.append(
                        (pltpu.async_copy(xs_h.at[colb.at[ck]], rbx, semx),
                         pltpu.async_copy(pwx_h.at[vib.at[ck]], vbx, semx)))
                sds = []
                for b in range(NBUF):
                    rbx, vbx, semx = bufs[b]
                    ck = k * NBUF + b
                    d1, d2 = gds[b]
                    d1.wait()
                    d2.wait()

                    def m_body(r, c3, rbx=rbx, vbx=vbx):
                        for u in range(2):
                            for q in range(FH // 16):
                                rbx[2 * r + u, pl.ds(q * 16, 16)] = (
                                    rbx[2 * r + u, pl.ds(q * 16, 16)]
                                    * vbx[2 * r + u, pl.ds(q * 16, 16)])
                        return c3
                    lax.fori_loop(0, 64, m_body, 0)
                    sds.append(pltpu.async_copy(
                        rbx, acc.at[rowb.at[ck]], semx, add=True))
                for d in sds:
                    d.wait()
                return c
            lax.fori_loop(0, 32 // NBUF, k_body, 0)
            return c0
        lax.fori_loop(0, 4, macro, 0)
        plsc.subcore_barrier()

        pltpu.sync_copy(acc.at[pl.ds(sid * 1024, 1024)],
                        chf_h.at[pl.ds(sid * 1024, 1024)])
        pltpu.sync_copy(disp_h.at[pl.ds(sid * 32, 32)], dispb)
        plsc.subcore_barrier()

        def gk_body(k, c):
            gds = []
            for b in range(NBUF):
                rbx, _, semx = bufs[b]
                ck = k * NBUF + b
                gds.append(pltpu.async_copy(chf_h.at[dispb.at[ck]],
                                            rbx, semx))
            for b in range(NBUF):
                rbx, _, _ = bufs[b]
                ck = k * NBUF + b
                gds[b].wait()
                pltpu.sync_copy(
                    rbx, chg_h.at[pl.ds(sid * 4096 + ck * 128, 128)])
            return c
        lax.fori_loop(0, 32 // NBUF, gk_body, 0)

    pl.when(cid == 0)(lambda: process(xsl_h, chgl_h, chfl_h))
    pl.when(cid == 1)(lambda: process(xsr_h, chgr_h, chfr_h))


@functools.partial(jax.jit, static_argnums=())
def _spmm_gather(rows2, cols2, vidx2, xsl, xsr, dispi, pw_pad):
    mesh = plsc.VectorSubcoreMesh(core_axis_name="c", subcore_axis_name="s")
    f = pl.kernel(
        _spmm_body,
        out_type=[jax.ShapeDtypeStruct((NDISP, FH), jnp.float32),
                  jax.ShapeDtypeStruct((NDISP, FH), jnp.float32),
                  jax.ShapeDtypeStruct((SEC, FH), jnp.float32),
                  jax.ShapeDtypeStruct((SEC, FH), jnp.float32)],
        mesh=mesh,
        scratch_types=(
            [pltpu.VMEM((32, 128), jnp.int32)] * 4
            + [pltpu.VMEM((128, FH), jnp.float32)] * 4
            + [pltpu.VMEM_SHARED((SEC, FH), jnp.float32)]
            + [pltpu.SemaphoreType.DMA] * 2),
        compiler_params=pltpu.CompilerParams(needs_layout_passes=False,
                                             use_tc_tiling_on_sc=False),
    )
    chgl, chgr, _, _ = f(rows2, cols2, vidx2, xsl, xsr, dispi, pw_pad)
    return chgl, chgr


def _seg_body(exp_h, disp_h, zeros_h, out_h, ibuf, dbuf, sacc, sem):
    cid = lax.axis_index("c")
    sid = lax.axis_index("s")

    @pl.when(cid == 0)
    def _():
        pltpu.sync_copy(zeros_h.at[pl.ds(sid * 1024, 1024)],
                        sacc.at[pl.ds(sid * 1024, 1024)])
        plsc.subcore_barrier()

        def sc_body(k, c):
            gbase = sid * 4096 + k * 128
            pltpu.sync_copy(disp_h.at[pl.ds(gbase, 128)], ibuf.at[0])
            pltpu.sync_copy(exp_h.at[pl.ds(gbase, 128)], dbuf)
            pltpu.sync_copy(dbuf, sacc.at[ibuf.at[0]], add=True)
            return c
        lax.fori_loop(0, 32, sc_body, 0)
        plsc.subcore_barrier()
        pltpu.sync_copy(sacc.at[pl.ds(sid * 1024, 1024)],
                        out_h.at[pl.ds(sid * 1024, 1024)])


def _segsum(exp8, dispi, zeros8):
    mesh = plsc.VectorSubcoreMesh(core_axis_name="c", subcore_axis_name="s")
    f = pl.kernel(
        _seg_body,
        out_type=jax.ShapeDtypeStruct((SEC, 8), jnp.float32),
        mesh=mesh,
        scratch_types=[
            pltpu.VMEM((1, 128), jnp.int32),
            pltpu.VMEM((128, 8), jnp.float32),
            pltpu.VMEM_SHARED((SEC, 8), jnp.float32),
            pltpu.SemaphoreType.DMA,
        ],
        compiler_params=pltpu.CompilerParams(needs_layout_passes=False,
                                             use_tc_tiling_on_sc=False),
    )
    return f(exp8, dispi, zeros8)


def _mlp_body(chgl_ref, chgr_ref, disp_ref, w1_ref, b1_ref, w2_ref, b2_ref,
              wout_ref, bout_ref, out_ref):
    x = jnp.concatenate(
        [chgl_ref[...].astype(jnp.bfloat16),
         chgr_ref[...].astype(jnp.bfloat16),
         disp_ref[...]], axis=1)
    h1 = jnp.dot(x, w1_ref[...], preferred_element_type=jnp.float32)
    h1 = h1 + b1_ref[...]
    h1 = jnp.where(h1 > 0, h1, jnp.exp(h1) - 1.0)
    h1 = h1.astype(jnp.bfloat16)
    h2 = jnp.dot(h1, w2_ref[...], preferred_element_type=jnp.float32)
    h2 = h2 + b2_ref[...]
    h2 = jnp.where(h2 > 0, h2, jnp.exp(h2) - 1.0)
    h2 = h2.astype(jnp.bfloat16)
    u = jnp.dot(h2, wout_ref[...], preferred_element_type=jnp.float32)
    u = u + bout_ref[...]
    out_ref[...] = jnp.exp(u[:, :8])


def _mlp_exp(chgl, chgr, disp, w1t, b1, w2t, b2, woutt, bout):
    n = disp.shape[0]
    grid = n // R
    return pl.pallas_call(
        _mlp_body,
        grid=(grid,),
        in_specs=[
            pl.BlockSpec((R, FH), lambda i: (i, 0)),
            pl.BlockSpec((R, FH), lambda i: (i, 0)),
            pl.BlockSpec((R, F), lambda i: (i, 0)),
            pl.BlockSpec((2 * F, 1024), lambda i: (0, 0)),
            pl.BlockSpec((1, 1024), lambda i: (0, 0)),
            pl.BlockSpec((1024, 1024), lambda i: (0, 0)),
            pl.BlockSpec((1, 1024), lambda i: (0, 0)),
            pl.BlockSpec((1024, 128), lambda i: (0, 0)),
            pl.BlockSpec((1, 128), lambda i: (0, 0)),
        ],
        out_specs=pl.BlockSpec((R, 8), lambda i: (i, 0)),
        out_shape=jax.ShapeDtypeStruct((n, 8), jnp.float32),
    )(chgl, chgr, disp, w1t, b1, w2t, b2, woutt, bout)


def kernel(section_length, item_size, cumsum_tril_value_indices,
           cumsum_tril_indices, Xs_clicked, disp_2d_split_sec_ind,
           disp_current_feature, W1, b1, W2, b2, W_out, b_out,
           position_weight):
    rows2 = cumsum_tril_indices[:, 0].reshape(NNZ // 128, 128)
    cols2 = cumsum_tril_indices[:, 1].reshape(NNZ // 128, 128)
    vidx2 = cumsum_tril_value_indices.reshape(NNZ // 128, 128)
    xsl = Xs_clicked[:, :FH]
    xsr = Xs_clicked[:, FH:]
    pw_pad = jnp.zeros((64,), jnp.float32).at[:50].set(position_weight)
    pw_exp = jnp.broadcast_to(pw_pad[:, None], (64, FH))
    dispi = disp_2d_split_sec_ind.astype(jnp.int32)
    disp2 = dispi.reshape(NDISP // 128, 128)

    chgl, chgr = _spmm_gather(rows2, cols2, vidx2, xsl, xsr, disp2, pw_exp)

    w1_hist = (W1[:, 0:F] + W1[:, F:2 * F] + W1[:, 2 * F:3 * F]
               + W1[:, 3 * F:4 * F])
    w1t = jnp.concatenate([w1_hist, W1[:, 4 * F:5 * F]], axis=1).T
    w1t = w1t.astype(jnp.bfloat16)
    w2t = W2.T.astype(jnp.bfloat16)
    woutt = jnp.zeros((1024, 128), jnp.bfloat16).at[:, 0].set(
        W_out[0, :].astype(jnp.bfloat16))
    boutv = jnp.zeros((1, 128), jnp.float32).at[0, 0].set(b_out[0])

    exp8 = _mlp_exp(chgl, chgr, disp_current_feature.astype(jnp.bfloat16),
                    w1t, b1.reshape(1, -1), w2t, b2.reshape(1, -1),
                    woutt, boutv)

    zeros8 = jnp.zeros((SEC, 8), jnp.float32)
    out8 = _segsum(exp8, dispi, zeros8)
    return out8[:, 0:1]
